# Initial kernel scaffold; baseline (speedup 1.0000x reference)
import jax
import jax.numpy as jnp
from jax.experimental import pallas as pl


def kernel(h_entity, x_link, edge_index_link, edge_weight_link, edge_index, W_edge, b_edge, hl_w0, hl_w1, hl_w2, hl_b, W_cau, b_cau, W_t, b_t, gcn_w0, gcn_w1, gcn_w2, gcn_b):
    raise NotImplementedError("write your pallas kernel here")


# speedup vs baseline: 1.4910x; 1.4910x over previous
import functools

import jax
import jax.numpy as jnp
from jax import lax
from jax.experimental import pallas as pl
from jax.experimental.pallas import tpu as pltpu
from jax.experimental.pallas import tpu_sc as plsc

N_NODES = 10000
N_EDGES = 320000
N_LL = 640000
D = 128
T = 16

NC = 2
NS = 16
L = 16


def _he_body(h_ref, wt_ref, bt_ref, out_ref):
    acc = h_ref[0] * wt_ref[0, 0]
    for t in range(1, T):
        acc = acc + h_ref[t] * wt_ref[t, 0]
    out_ref[...] = acc + bt_ref[0, 0]


def _t8_body(x_ref, m_ref, b_ref, out_ref):
    x = x_ref[...]
    acc = jnp.broadcast_to(b_ref[...], (x.shape[0], 8))
    for k in range(8):
        acc = acc + x[:, k:k + 1] * m_ref[k:k + 1, :]
    out_ref[...] = acc


def _combine_body(p_ref, out_ref):
    out_ref[...] = p_ref[0] + p_ref[1]


def _final_body(he_ref, h1_ref, h2p_ref, g0_ref, g1_ref, g2_ref, gb_ref, out_ref):
    f32 = jnp.float32
    acc = jnp.dot(he_ref[...], g0_ref[...], preferred_element_type=f32)
    acc = acc + jnp.dot(h1_ref[...], g1_ref[...], preferred_element_type=f32)
    h2 = h2p_ref[0] + h2p_ref[1]
    acc = acc + jnp.dot(h2, g2_ref[...], preferred_element_type=f32)
    out_ref[...] = acc + gb_ref[...]


_LCHUNK = 80
_ECHUNK = 80
_RCHUNK = 80


def _link_body(td0_hbm, td1_hbm, u0_hbm, u1_hbm, u2_hbm, u3_hbm,
               lsrc_hbm, ldst_hbm, lw_hbm, z1_hbm,
               w1_hbm, w2_hbm,
               vacc0, vacc1, vacc2, vacc3, wacc0, wacc1,
               li_v, di_v, lw_v, g_v, val_v, a_v, b_v, w1_v, w2_v, sem):
    c = lax.axis_index("c")
    s = lax.axis_index("s")
    vaccs = [vacc0, vacc1, vacc2, vacc3]

    zrows = N_EDGES // NS
    for acc in (vacc0, vacc1, vacc2, vacc3, wacc0, wacc1):
        pltpu.sync_copy(z1_hbm, acc.at[pl.ds(s * zrows, zrows)])
    plsc.subcore_barrier()

    links_per_tile = N_LL // NS
    nchunks1 = links_per_tile // _LCHUNK
    nv = _LCHUNK // L

    def round1(j, _):
        base = s * links_per_tile + j * _LCHUNK
        pltpu.sync_copy(lsrc_hbm.at[pl.ds(base, _LCHUNK)], li_v)
        pltpu.sync_copy(ldst_hbm.at[pl.ds(base, _LCHUNK)], di_v)
        pltpu.sync_copy(lw_hbm.at[pl.ds(base, _LCHUNK)], lw_v)
        for col, u_hbm in enumerate((u0_hbm, u1_hbm, u2_hbm, u3_hbm)):
            pltpu.async_copy(u_hbm.at[li_v], g_v, sem).wait()
            for jj in range(nv):
                sl = pl.ds(jj * L, L)
                val_v[sl] = g_v[sl] * lw_v[sl]
            pltpu.sync_copy(val_v, vaccs[col].at[di_v], add=True)
        return _

    lax.fori_loop(0, nchunks1, round1, 0)
    plsc.subcore_barrier()

    def round2(j, _):
        base = s * links_per_tile + j * _LCHUNK
        pltpu.sync_copy(lsrc_hbm.at[pl.ds(base, _LCHUNK)], li_v)
        pltpu.sync_copy(ldst_hbm.at[pl.ds(base, _LCHUNK)], di_v)
        pltpu.sync_copy(lw_hbm.at[pl.ds(base, _LCHUNK)], lw_v)
        for col, wa in ((2, wacc0), (3, wacc1)):
            pltpu.async_copy(vaccs[col].at[li_v], g_v, sem).wait()
            for jj in range(nv):
                sl = pl.ds(jj * L, L)
                val_v[sl] = g_v[sl] * lw_v[sl]
            pltpu.sync_copy(val_v, wa.at[di_v], add=True)
        return _

    lax.fori_loop(0, nchunks1, round2, 0)
    plsc.subcore_barrier()

    rows_per_tile = N_EDGES // (NC * NS)
    nchunks3 = rows_per_tile // _RCHUNK
    nr = _RCHUNK // L

    def fin(j, _):
        base = (c * NS + s) * rows_per_tile + j * _RCHUNK
        for td_hbm, va, wa, out_v, out_hbm in (
                (td0_hbm, vacc0, wacc0, w1_v, w1_hbm),
                (td1_hbm, vacc1, wacc1, w2_v, w2_hbm)):
            pltpu.sync_copy(td_hbm.at[pl.ds(base, _RCHUNK)], g_v)
            pltpu.sync_copy(va.at[pl.ds(base, _RCHUNK)], a_v)
            pltpu.sync_copy(wa.at[pl.ds(base, _RCHUNK)], b_v)
            for jj in range(nr):
                sl = pl.ds(jj * L, L)
                d = g_v[sl] - a_v[sl] + b_v[sl]
                out_v[sl] = 1.0 / (1.0 + jnp.exp(-d))
            pltpu.sync_copy(out_v, out_hbm.at[pl.ds(base, _RCHUNK)])
        return _

    lax.fori_loop(0, nchunks3, fin, 0)


def _gcn_body(table_hbm, nsrc_hbm, ndst_hbm, w_hbm, z128_hbm,
              part_hbm,
              acc, si_v, di_v, w_v, rows_v, sem):
    c = lax.axis_index("c")
    s = lax.axis_index("s")
    iota = lax.iota(jnp.int32, L)

    zrows = N_NODES // NS
    pltpu.sync_copy(z128_hbm, acc.at[pl.ds(s * zrows, zrows)])
    plsc.subcore_barrier()

    edges_per_tile = N_EDGES // (NC * NS)
    nchunks = edges_per_tile // _ECHUNK

    def chunk(j, _):
        base = (c * NS + s) * edges_per_tile + j * _ECHUNK
        pltpu.sync_copy(nsrc_hbm.at[pl.ds(base, _ECHUNK)], si_v)
        pltpu.sync_copy(ndst_hbm.at[pl.ds(base, _ECHUNK)], di_v)
        pltpu.sync_copy(w_hbm.at[pl.ds(base, _ECHUNK)], w_v)
        pltpu.async_copy(table_hbm.at[si_v], rows_v, sem).wait()

        def scale(i, _2):
            wsplat = plsc.load_gather(w_v, [iota * 0 + i])
            for k in range(D // L):
                rows_v[i, pl.ds(k * L, L)] = rows_v[i, pl.ds(k * L, L)] * wsplat
            return _2

        lax.fori_loop(0, _ECHUNK, scale, 0)
        pltpu.sync_copy(rows_v, acc.at[di_v], add=True)
        return _

    lax.fori_loop(0, nchunks, chunk, 0)
    plsc.subcore_barrier()
    pltpu.sync_copy(acc.at[pl.ds(s * zrows, zrows)],
                    part_hbm.at[c, pl.ds(s * zrows, zrows)])


def _sc_mesh():
    return plsc.VectorSubcoreMesh(core_axis_name="c", subcore_axis_name="s",
                                  num_cores=NC, num_subcores=NS)


@functools.partial(jax.jit, static_argnames=())
def _link_call(td0, td1, u0, u1, u2, u3, lsrc, ldst, lw):
    z1 = jnp.zeros((N_EDGES // NS,), jnp.float32)
    f = pl.kernel(
        _link_body,
        out_type=(jax.ShapeDtypeStruct((N_EDGES,), jnp.float32),
                  jax.ShapeDtypeStruct((N_EDGES,), jnp.float32)),
        mesh=_sc_mesh(),
        compiler_params=pltpu.CompilerParams(needs_layout_passes=False, use_tc_tiling_on_sc=False),
        scratch_types=[
            pltpu.VMEM_SHARED((N_EDGES,), jnp.float32),
            pltpu.VMEM_SHARED((N_EDGES,), jnp.float32),
            pltpu.VMEM_SHARED((N_EDGES,), jnp.float32),
            pltpu.VMEM_SHARED((N_EDGES,), jnp.float32),
            pltpu.VMEM_SHARED((N_EDGES,), jnp.float32),
            pltpu.VMEM_SHARED((N_EDGES,), jnp.float32),
            pltpu.VMEM((_LCHUNK,), jnp.int32),
            pltpu.VMEM((_LCHUNK,), jnp.int32),
            pltpu.VMEM((_LCHUNK,), jnp.float32),
            pltpu.VMEM((_LCHUNK,), jnp.float32),
            pltpu.VMEM((_LCHUNK,), jnp.float32),
            pltpu.VMEM((_RCHUNK,), jnp.float32),
            pltpu.VMEM((_RCHUNK,), jnp.float32),
            pltpu.VMEM((_RCHUNK,), jnp.float32),
            pltpu.VMEM((_RCHUNK,), jnp.float32),
            pltpu.SemaphoreType.DMA,
        ],
    )
    return f(td0, td1, u0, u1, u2, u3, lsrc, ldst, lw, z1)


@jax.jit
def _gcn_call(table, nsrc, ndst, w):
    z128 = jnp.zeros((N_NODES // NS, D), jnp.float32)
    f = pl.kernel(
        _gcn_body,
        out_type=jax.ShapeDtypeStruct((NC, N_NODES, D), jnp.float32),
        mesh=_sc_mesh(),
        compiler_params=pltpu.CompilerParams(needs_layout_passes=False, use_tc_tiling_on_sc=False),
        scratch_types=[
            pltpu.VMEM_SHARED((N_NODES, D), jnp.float32),
            pltpu.VMEM((_ECHUNK,), jnp.int32),
            pltpu.VMEM((_ECHUNK,), jnp.int32),
            pltpu.VMEM((_ECHUNK,), jnp.float32),
            pltpu.VMEM((_ECHUNK, D), jnp.float32),
            pltpu.SemaphoreType.DMA,
        ],
    )
    return f(table, nsrc, ndst, w, z128)


def kernel(h_entity, x_link, edge_index_link, edge_weight_link, edge_index,
           W_edge, b_edge, hl_w0, hl_w1, hl_w2, hl_b, W_cau, b_cau, W_t, b_t,
           gcn_w0, gcn_w1, gcn_w2, gcn_b):
    f32 = jnp.float32
    lsrc = edge_index_link[0]
    ldst = edge_index_link[1]
    nsrc = edge_index[0]
    ndst = edge_index[1]

    Wd = jnp.stack([W_cau[:, 2] - W_cau[:, 3], W_cau[:, 4] - W_cau[:, 5]], axis=1)
    bd = jnp.stack([b_cau[2] - b_cau[3], b_cau[4] - b_cau[5]])
    A0, A1, A2 = hl_w0 @ Wd, hl_w1 @ Wd, hl_w2 @ Wd
    C0 = A0 + A1 + A2
    C12 = jnp.concatenate([A1 + 2.0 * A2, 0.5 * A2], axis=1)
    M8 = jnp.concatenate([W_edge @ C0, W_edge @ C12,
                          jnp.zeros((8, 2), f32)], axis=1)
    b8 = jnp.concatenate([b_edge @ C0 + hl_b @ Wd + bd, b_edge @ C12,
                          jnp.zeros((2,), f32)])[None, :]

    BE = 4000
    t8 = pl.pallas_call(
        _t8_body,
        grid=(N_EDGES // BE,),
        in_specs=[pl.BlockSpec((BE, 8), lambda i: (i, 0)),
                  pl.BlockSpec((8, 8), lambda i: (0, 0)),
                  pl.BlockSpec((1, 8), lambda i: (0, 0))],
        out_specs=pl.BlockSpec((BE, 8), lambda i: (i, 0)),
        out_shape=jax.ShapeDtypeStruct((N_EDGES, 8), f32),
    )(x_link, M8, b8)

    BN = 1000
    he = pl.pallas_call(
        _he_body,
        grid=(N_NODES // BN,),
        in_specs=[pl.BlockSpec((T, BN, D), lambda i: (0, i, 0)),
                  pl.BlockSpec((T, 1), lambda i: (0, 0)),
                  pl.BlockSpec((1, 1), lambda i: (0, 0))],
        out_specs=pl.BlockSpec((BN, D), lambda i: (i, 0)),
        out_shape=jax.ShapeDtypeStruct((N_NODES, D), f32),
    )(h_entity, W_t, b_t[:, None])

    w1, w2 = _link_call(t8[:, 0], t8[:, 1], t8[:, 2], t8[:, 3], t8[:, 4],
                        t8[:, 5], lsrc, ldst, edge_weight_link)

    h1p = _gcn_call(he, nsrc, ndst, w1)

    BC = 1000
    h1 = pl.pallas_call(
        _combine_body,
        grid=(N_NODES // BC,),
        in_specs=[pl.BlockSpec((2, BC, D), lambda i: (0, i, 0))],
        out_specs=pl.BlockSpec((BC, D), lambda i: (i, 0)),
        out_shape=jax.ShapeDtypeStruct((N_NODES, D), f32),
    )(h1p)

    h2p = _gcn_call(h1, nsrc, ndst, w2)

    BF = 1000
    out2 = pl.pallas_call(
        _final_body,
        grid=(N_NODES // BF,),
        in_specs=[pl.BlockSpec((BF, D), lambda i: (i, 0)),
                  pl.BlockSpec((BF, D), lambda i: (i, 0)),
                  pl.BlockSpec((2, BF, D), lambda i: (0, i, 0)),
                  pl.BlockSpec((D, D), lambda i: (0, 0)),
                  pl.BlockSpec((D, D), lambda i: (0, 0)),
                  pl.BlockSpec((D, D), lambda i: (0, 0)),
                  pl.BlockSpec((1, D), lambda i: (0, 0))],
        out_specs=pl.BlockSpec((BF, D), lambda i: (i, 0)),
        out_shape=jax.ShapeDtypeStruct((N_NODES, D), f32),
    )(he, h1, h2p, gcn_w0, gcn_w1, gcn_w2, gcn_b[None, :])
    return out2


# speedup vs baseline: 3.9064x; 2.6199x over previous
import functools

import jax
import jax.numpy as jnp
from jax import lax
from jax.experimental import pallas as pl
from jax.experimental.pallas import tpu as pltpu
from jax.experimental.pallas import tpu_sc as plsc

N_NODES = 10000
N_EDGES = 320000
N_LL = 640000
D = 128
T = 16

NC = 2
NS = 16
L = 16


def _he_body(h_ref, wt_ref, bt_ref, out_ref):
    acc = h_ref[0] * wt_ref[0, 0]
    for t in range(1, T):
        acc = acc + h_ref[t] * wt_ref[t, 0]
    out_ref[...] = acc + bt_ref[0, 0]


def _t8_body(x_ref, m_ref, b_ref, out_ref):
    x = x_ref[...]
    acc = jnp.broadcast_to(b_ref[...], (x.shape[0], 8))
    for k in range(8):
        acc = acc + x[:, k:k + 1] * m_ref[k:k + 1, :]
    out_ref[...] = acc


def _combine_body(p_ref, out_ref):
    out_ref[...] = p_ref[0] + p_ref[1]


def _final_body(he_ref, h1_ref, h2p_ref, g0_ref, g1_ref, g2_ref, gb_ref, out_ref):
    f32 = jnp.float32
    acc = jnp.dot(he_ref[...], g0_ref[...], preferred_element_type=f32)
    acc = acc + jnp.dot(h1_ref[...], g1_ref[...], preferred_element_type=f32)
    h2 = h2p_ref[0] + h2p_ref[1]
    acc = acc + jnp.dot(h2, g2_ref[...], preferred_element_type=f32)
    out_ref[...] = acc + gb_ref[...]


_LCHUNK = 4000
_ECHUNK = 80
_RCHUNK = 2000


def _lprop_body(u0_hbm, u1_hbm, u2_hbm, u3_hbm,
                lsrc_hbm, ldst_hbm, lw_hbm, z1_hbm,
                p0_hbm, p1_hbm, p2_hbm, p3_hbm,
                acc0, acc1, acc2, acc3,
                li_v, di_v, lw_v, g0_v, g1_v, g2_v, g3_v,
                v0_v, v1_v, v2_v, v3_v, sem, sem2):
    c = lax.axis_index("c")
    s = lax.axis_index("s")
    cols = [(u, p, a, g, v) for u, p, a, g, v in
            ((u0_hbm, p0_hbm, acc0, g0_v, v0_v),
             (u1_hbm, p1_hbm, acc1, g1_v, v1_v),
             (u2_hbm, p2_hbm, acc2, g2_v, v2_v),
             (u3_hbm, p3_hbm, acc3, g3_v, v3_v)) if u is not None]

    zrows = N_EDGES // NS
    for col in cols:
        pltpu.sync_copy(z1_hbm, col[2].at[pl.ds(s * zrows, zrows)])
    plsc.subcore_barrier()

    links_per_tile = N_LL // (NC * NS)
    nchunks = links_per_tile // _LCHUNK
    nv = _LCHUNK // L

    def chunk(j, carry):
        base = (c * NS + s) * links_per_tile + j * _LCHUNK
        pltpu.sync_copy(lsrc_hbm.at[pl.ds(base, _LCHUNK)], li_v)
        pltpu.sync_copy(ldst_hbm.at[pl.ds(base, _LCHUNK)], di_v)
        pltpu.sync_copy(lw_hbm.at[pl.ds(base, _LCHUNK)], lw_v)
        descs = [pltpu.async_copy(col[0].at[li_v], col[3], sem)
                 for col in cols]
        sdescs = []
        for k, col in enumerate(cols):
            a, g, v = col[2], col[3], col[4]
            descs[k].wait()

            def mul(jj, c2, g=g, v=v):
                sl = pl.ds(jj * L, L)
                v[sl] = g[sl] * lw_v[sl]
                return c2

            lax.fori_loop(0, nv, mul, 0)
            sdescs.append(pltpu.async_copy(v, a.at[di_v], sem2, add=True))
        for dd in sdescs:
            dd.wait()
        return carry

    lax.fori_loop(0, nchunks, chunk, 0)
    plsc.subcore_barrier()
    for col in cols:
        pltpu.sync_copy(col[2].at[pl.ds(s * zrows, zrows)],
                        col[1].at[c, pl.ds(s * zrows, zrows)])


def _vcomb4_body(p0, p1, p2, p3, o0, o1, o2, o3):
    o0[...] = p0[0] + p0[1]
    o1[...] = p1[0] + p1[1]
    o2[...] = p2[0] + p2[1]
    o3[...] = p3[0] + p3[1]


def _gate2_body(td0, v0, q0, td1, v1, q1, w1o, w2o):
    d0 = td0[...] - v0[...] + q0[0] + q0[1]
    w1o[...] = 1.0 / (1.0 + jnp.exp(-d0))
    d1 = td1[...] - v1[...] + q1[0] + q1[1]
    w2o[...] = 1.0 / (1.0 + jnp.exp(-d1))


def _gcn_body(table_hbm, nsrc_hbm, ndst_hbm, w_hbm, z128_hbm,
              part_hbm,
              acc, si0_v, si1_v, di0_v, di1_v, w0_v, w1_v,
              rows0_v, rows1_v, sem0, sem1):
    c = lax.axis_index("c")
    s = lax.axis_index("s")
    iota = lax.iota(jnp.int32, L)
    si = (si0_v, si1_v)
    di = (di0_v, di1_v)
    wv = (w0_v, w1_v)
    rows = (rows0_v, rows1_v)
    sems = (sem0, sem1)

    zrows = N_NODES // NS
    pltpu.sync_copy(z128_hbm, acc.at[pl.ds(s * zrows, zrows)])
    plsc.subcore_barrier()

    edges_per_tile = N_EDGES // (NC * NS)
    nchunks = edges_per_tile // _ECHUNK
    base0 = (c * NS + s) * edges_per_tile

    def load_idx(j, b):
        pltpu.sync_copy(nsrc_hbm.at[pl.ds(base0 + j * _ECHUNK, _ECHUNK)], si[b])
        pltpu.sync_copy(ndst_hbm.at[pl.ds(base0 + j * _ECHUNK, _ECHUNK)], di[b])
        pltpu.sync_copy(w_hbm.at[pl.ds(base0 + j * _ECHUNK, _ECHUNK)], wv[b])

    load_idx(0, 0)
    desc = {0: pltpu.async_copy(table_hbm.at[si[0]], rows[0], sems[0])}
    for j in range(nchunks):
        b = j % 2
        if j + 1 < nchunks:
            nb = (j + 1) % 2
            load_idx(j + 1, nb)
            desc[nb] = pltpu.async_copy(table_hbm.at[si[nb]], rows[nb], sems[nb])
        desc[b].wait()

        def scale(i, _2, b=b):
            wsplat = plsc.load_gather(wv[b], [iota * 0 + i])
            for k in range(D // L):
                rows[b][i, pl.ds(k * L, L)] = rows[b][i, pl.ds(k * L, L)] * wsplat
            return _2

        lax.fori_loop(0, _ECHUNK, scale, 0)
        pltpu.sync_copy(rows[b], acc.at[di[b]], add=True)

    plsc.subcore_barrier()
    pltpu.sync_copy(acc.at[pl.ds(s * zrows, zrows)],
                    part_hbm.at[c, pl.ds(s * zrows, zrows)])


def _sc_mesh():
    return plsc.VectorSubcoreMesh(core_axis_name="c", subcore_axis_name="s",
                                  num_cores=NC, num_subcores=NS)


def _lprop_call(ucols, lsrc, ldst, lw):
    n = len(ucols)
    pad = 4 - n
    z1 = jnp.zeros((N_EDGES // NS,), jnp.float32)

    def body(*refs):
        us = list(refs[0:n]) + [None] * pad
        lsrc_r, ldst_r, lw_r, z1_r = refs[n:n + 4]
        ps = list(refs[n + 4:n + 4 + n]) + [None] * pad
        rest = refs[n + 4 + n:]
        accs = list(rest[0:n]) + [None] * pad
        li_v, di_v, lw_v = rest[n:n + 3]
        gs = list(rest[n + 3:n + 3 + n]) + [None] * pad
        vs = list(rest[n + 3 + n:n + 3 + 2 * n]) + [None] * pad
        sem, sem2 = rest[n + 3 + 2 * n:]
        _lprop_body(us[0], us[1], us[2], us[3], lsrc_r, ldst_r, lw_r, z1_r,
                    ps[0], ps[1], ps[2], ps[3], accs[0], accs[1], accs[2],
                    accs[3], li_v, di_v, lw_v, gs[0], gs[1], gs[2], gs[3],
                    vs[0], vs[1], vs[2], vs[3], sem, sem2)

    f = pl.kernel(
        body,
        out_type=tuple(jax.ShapeDtypeStruct((NC, N_EDGES), jnp.float32)
                       for _ in range(n)),
        mesh=_sc_mesh(),
        compiler_params=pltpu.CompilerParams(needs_layout_passes=False, use_tc_tiling_on_sc=False),
        scratch_types=(
            [pltpu.VMEM_SHARED((N_EDGES,), jnp.float32)] * n +
            [pltpu.VMEM((_LCHUNK,), jnp.int32),
             pltpu.VMEM((_LCHUNK,), jnp.int32),
             pltpu.VMEM((_LCHUNK,), jnp.float32)] +
            [pltpu.VMEM((_LCHUNK,), jnp.float32)] * (2 * n) +
            [pltpu.SemaphoreType.DMA, pltpu.SemaphoreType.DMA]
        ),
    )
    return f(*ucols, lsrc, ldst, lw, z1)


@jax.jit
def _gcn_call(table, nsrc, ndst, w):
    z128 = jnp.zeros((N_NODES // NS, D), jnp.float32)
    f = pl.kernel(
        _gcn_body,
        out_type=jax.ShapeDtypeStruct((NC, N_NODES, D), jnp.float32),
        mesh=_sc_mesh(),
        compiler_params=pltpu.CompilerParams(needs_layout_passes=False, use_tc_tiling_on_sc=False),
        scratch_types=[
            pltpu.VMEM_SHARED((N_NODES, D), jnp.float32),
            pltpu.VMEM((_ECHUNK,), jnp.int32),
            pltpu.VMEM((_ECHUNK,), jnp.int32),
            pltpu.VMEM((_ECHUNK,), jnp.int32),
            pltpu.VMEM((_ECHUNK,), jnp.int32),
            pltpu.VMEM((_ECHUNK,), jnp.float32),
            pltpu.VMEM((_ECHUNK,), jnp.float32),
            pltpu.VMEM((_ECHUNK, D), jnp.float32),
            pltpu.VMEM((_ECHUNK, D), jnp.float32),
            pltpu.SemaphoreType.DMA,
            pltpu.SemaphoreType.DMA,
        ],
    )
    return f(table, nsrc, ndst, w, z128)


def kernel(h_entity, x_link, edge_index_link, edge_weight_link, edge_index,
           W_edge, b_edge, hl_w0, hl_w1, hl_w2, hl_b, W_cau, b_cau, W_t, b_t,
           gcn_w0, gcn_w1, gcn_w2, gcn_b):
    f32 = jnp.float32
    lsrc = edge_index_link[0]
    ldst = edge_index_link[1]
    nsrc = edge_index[0]
    ndst = edge_index[1]

    Wd = jnp.stack([W_cau[:, 2] - W_cau[:, 3], W_cau[:, 4] - W_cau[:, 5]], axis=1)
    bd = jnp.stack([b_cau[2] - b_cau[3], b_cau[4] - b_cau[5]])
    A0, A1, A2 = hl_w0 @ Wd, hl_w1 @ Wd, hl_w2 @ Wd
    C0 = A0 + A1 + A2
    C12 = jnp.concatenate([A1 + 2.0 * A2, 0.5 * A2], axis=1)
    M8 = jnp.concatenate([W_edge @ C0, W_edge @ C12,
                          jnp.zeros((8, 2), f32)], axis=1)
    b8 = jnp.concatenate([b_edge @ C0 + hl_b @ Wd + bd, b_edge @ C12,
                          jnp.zeros((2,), f32)])[None, :]

    BE = 4000
    t8 = pl.pallas_call(
        _t8_body,
        grid=(N_EDGES // BE,),
        in_specs=[pl.BlockSpec((BE, 8), lambda i: (i, 0)),
                  pl.BlockSpec((8, 8), lambda i: (0, 0)),
                  pl.BlockSpec((1, 8), lambda i: (0, 0))],
        out_specs=pl.BlockSpec((BE, 8), lambda i: (i, 0)),
        out_shape=jax.ShapeDtypeStruct((N_EDGES, 8), f32),
    )(x_link, M8, b8)

    BN = 1000
    he = pl.pallas_call(
        _he_body,
        grid=(N_NODES // BN,),
        in_specs=[pl.BlockSpec((T, BN, D), lambda i: (0, i, 0)),
                  pl.BlockSpec((T, 1), lambda i: (0, 0)),
                  pl.BlockSpec((1, 1), lambda i: (0, 0))],
        out_specs=pl.BlockSpec((BN, D), lambda i: (i, 0)),
        out_shape=jax.ShapeDtypeStruct((N_NODES, D), f32),
    )(h_entity, W_t, b_t[:, None])

    lw = edge_weight_link
    pv = _lprop_call([t8[:, 2], t8[:, 3], t8[:, 4], t8[:, 5]],
                     lsrc, ldst, lw)
    v0c, v1c, v2c, v3c = pl.pallas_call(
        _vcomb4_body,
        out_shape=tuple(jax.ShapeDtypeStruct((N_EDGES,), f32)
                        for _ in range(4)),
    )(*pv)
    q = _lprop_call([v2c, v3c], lsrc, ldst, lw)
    w1, w2 = pl.pallas_call(
        _gate2_body,
        out_shape=(jax.ShapeDtypeStruct((N_EDGES,), f32),
                   jax.ShapeDtypeStruct((N_EDGES,), f32)),
    )(t8[:, 0], v0c, q[0], t8[:, 1], v1c, q[1])

    h1p = _gcn_call(he, nsrc, ndst, w1)

    BC = 1000
    h1 = pl.pallas_call(
        _combine_body,
        grid=(N_NODES // BC,),
        in_specs=[pl.BlockSpec((2, BC, D), lambda i: (0, i, 0))],
        out_specs=pl.BlockSpec((BC, D), lambda i: (i, 0)),
        out_shape=jax.ShapeDtypeStruct((N_NODES, D), f32),
    )(h1p)

    h2p = _gcn_call(h1, nsrc, ndst, w2)

    BF = 1000
    out2 = pl.pallas_call(
        _final_body,
        grid=(N_NODES // BF,),
        in_specs=[pl.BlockSpec((BF, D), lambda i: (i, 0)),
                  pl.BlockSpec((BF, D), lambda i: (i, 0)),
                  pl.BlockSpec((2, BF, D), lambda i: (0, i, 0)),
                  pl.BlockSpec((D, D), lambda i: (0, 0)),
                  pl.BlockSpec((D, D), lambda i: (0, 0)),
                  pl.BlockSpec((D, D), lambda i: (0, 0)),
                  pl.BlockSpec((1, D), lambda i: (0, 0))],
        out_specs=pl.BlockSpec((BF, D), lambda i: (i, 0)),
        out_shape=jax.ShapeDtypeStruct((N_NODES, D), f32),
    )(he, h1, h2p, gcn_w0, gcn_w1, gcn_w2, gcn_b[None, :])
    return out2


# speedup vs baseline: 4.4962x; 1.1510x over previous
import functools

import jax
import jax.numpy as jnp
from jax import lax
from jax.experimental import pallas as pl
from jax.experimental.pallas import tpu as pltpu
from jax.experimental.pallas import tpu_sc as plsc

N_NODES = 10000
N_EDGES = 320000
N_LL = 640000
D = 128
T = 16

NC = 2
NS = 16
L = 16


def _he_body(h_ref, wt_ref, bt_ref, out_ref):
    acc = h_ref[0] * wt_ref[0, 0]
    for t in range(1, T):
        acc = acc + h_ref[t] * wt_ref[t, 0]
    out_ref[...] = acc + bt_ref[0, 0]


def _t8_body(x_ref, m_ref, b_ref, out_ref):
    x = x_ref[...]
    acc = jnp.broadcast_to(b_ref[...], (x.shape[0], 8))
    for k in range(8):
        acc = acc + x[:, k:k + 1] * m_ref[k:k + 1, :]
    out_ref[...] = acc


def _combine_body(p_ref, out_ref):
    out_ref[...] = p_ref[0] + p_ref[1]


def _final_body(he_ref, h1_ref, h2p_ref, g0_ref, g1_ref, g2_ref, gb_ref, out_ref):
    f32 = jnp.float32
    acc = jnp.dot(he_ref[...], g0_ref[...], preferred_element_type=f32)
    acc = acc + jnp.dot(h1_ref[...], g1_ref[...], preferred_element_type=f32)
    h2 = h2p_ref[0] + h2p_ref[1]
    acc = acc + jnp.dot(h2, g2_ref[...], preferred_element_type=f32)
    out_ref[...] = acc + gb_ref[...]


_LCHUNK = 4000
_ECHUNK = 80
_RCHUNK = 2000


def _lprop_body(u0_hbm, u1_hbm, u2_hbm, u3_hbm,
                lsrc_hbm, ldst_hbm, lw_hbm, z1_hbm,
                p0_hbm, p1_hbm, p2_hbm, p3_hbm,
                acc0, acc1, acc2, acc3,
                li_v, di_v, lw_v, g0_v, g1_v, g2_v, g3_v,
                v0_v, v1_v, v2_v, v3_v, sem, sem2, pairwise=False):
    c = lax.axis_index("c")
    s = lax.axis_index("s")
    cols = [(u, p, a, g, v) for u, p, a, g, v in
            ((u0_hbm, p0_hbm, acc0, g0_v, v0_v),
             (u1_hbm, p1_hbm, acc1, g1_v, v1_v),
             (u2_hbm, p2_hbm, acc2, g2_v, v2_v),
             (u3_hbm, p3_hbm, acc3, g3_v, v3_v)) if u is not None]

    zrows = N_EDGES // NS
    for col in cols:
        if col[2] is not None:
            pltpu.sync_copy(z1_hbm, col[2].at[pl.ds(s * zrows, zrows)])
    plsc.subcore_barrier()

    links_per_tile = N_LL // (NC * NS)
    nchunks = links_per_tile // _LCHUNK
    nv = _LCHUNK // L

    def chunk(j, carry):
        base = (c * NS + s) * links_per_tile + j * _LCHUNK
        pltpu.sync_copy(lsrc_hbm.at[pl.ds(base, _LCHUNK)], li_v)
        pltpu.sync_copy(ldst_hbm.at[pl.ds(base, _LCHUNK)], di_v)
        pltpu.sync_copy(lw_hbm.at[pl.ds(base, _LCHUNK)], lw_v)
        descs = [pltpu.async_copy(col[0].at[li_v], col[3], sem)
                 for col in cols]
        sdescs = []
        if pairwise:
            for k in range(len(cols) // 2):
                ca, cb = cols[2 * k], cols[2 * k + 1]
                descs[2 * k].wait()
                descs[2 * k + 1].wait()

                def mul(jj, c2, ga=ca[3], gb=cb[3], v=ca[4]):
                    sl = pl.ds(jj * L, L)
                    v[sl] = (ga[sl] + gb[sl]) * lw_v[sl]
                    return c2

                lax.fori_loop(0, nv, mul, 0)
                sdescs.append(pltpu.async_copy(ca[4], ca[2].at[di_v], sem2,
                                               add=True))
        else:
            for k, col in enumerate(cols):
                a, g, v = col[2], col[3], col[4]
                descs[k].wait()

                def mul(jj, c2, g=g, v=v):
                    sl = pl.ds(jj * L, L)
                    v[sl] = g[sl] * lw_v[sl]
                    return c2

                lax.fori_loop(0, nv, mul, 0)
                sdescs.append(pltpu.async_copy(v, a.at[di_v], sem2, add=True))
        for dd in sdescs:
            dd.wait()
        return carry

    lax.fori_loop(0, nchunks, chunk, 0)
    plsc.subcore_barrier()
    for col in cols:
        if col[2] is not None:
            pltpu.sync_copy(col[2].at[pl.ds(s * zrows, zrows)],
                            col[1].at[c, pl.ds(s * zrows, zrows)])


def _gate2_body(td0, v0, q0, td1, v1, q1, w1o, w2o):
    d0 = td0[...] - (v0[0] + v0[1]) + q0[0] + q0[1]
    w1o[...] = 1.0 / (1.0 + jnp.exp(-d0))
    d1 = td1[...] - (v1[0] + v1[1]) + q1[0] + q1[1]
    w2o[...] = 1.0 / (1.0 + jnp.exp(-d1))


def _gcn_body(table_hbm, nsrc_hbm, ndst_hbm, w_hbm, z128_hbm,
              part_hbm,
              acc, sib_v, dib_v, wb_v,
              rows0_v, rows1_v, sem0, sem1):
    c = lax.axis_index("c")
    s = lax.axis_index("s")
    iota = lax.iota(jnp.int32, L)
    rows = (rows0_v, rows1_v)
    sems = (sem0, sem1)

    zrows = N_NODES // NS
    pltpu.sync_copy(z128_hbm, acc.at[pl.ds(s * zrows, zrows)])
    plsc.subcore_barrier()

    edges_per_tile = N_EDGES // (NC * NS)
    nchunks = edges_per_tile // _ECHUNK
    base0 = (c * NS + s) * edges_per_tile

    pltpu.sync_copy(nsrc_hbm.at[pl.ds(base0, edges_per_tile)], sib_v)
    pltpu.sync_copy(ndst_hbm.at[pl.ds(base0, edges_per_tile)], dib_v)
    pltpu.sync_copy(w_hbm.at[pl.ds(base0, edges_per_tile)], wb_v)

    def gather(j, b):
        return pltpu.async_copy(
            table_hbm.at[sib_v.at[pl.ds(j * _ECHUNK, _ECHUNK)]], rows[b],
            sems[b])

    desc = {0: gather(0, 0)}
    for j in range(nchunks):
        b = j % 2
        if j + 1 < nchunks:
            nb = (j + 1) % 2
            desc[nb] = gather(j + 1, nb)
        desc[b].wait()

        def scale(i, _2, j=j, b=b):
            wsplat = plsc.load_gather(wb_v, [iota * 0 + (j * _ECHUNK + i)])
            for k in range(D // L):
                rows[b][i, pl.ds(k * L, L)] = rows[b][i, pl.ds(k * L, L)] * wsplat
            return _2

        lax.fori_loop(0, _ECHUNK, scale, 0)
        pltpu.sync_copy(rows[b], acc.at[dib_v.at[pl.ds(j * _ECHUNK, _ECHUNK)]],
                        add=True)

    plsc.subcore_barrier()
    pltpu.sync_copy(acc.at[pl.ds(s * zrows, zrows)],
                    part_hbm.at[c, pl.ds(s * zrows, zrows)])


def _sc_mesh():
    return plsc.VectorSubcoreMesh(core_axis_name="c", subcore_axis_name="s",
                                  num_cores=NC, num_subcores=NS)


def _lprop_call(ucols, lsrc, ldst, lw, pairwise=False):
    n_in = len(ucols)
    n_out = n_in // 2 if pairwise else n_in
    pad_in = 4 - n_in
    z1 = jnp.zeros((N_EDGES // NS,), jnp.float32)

    def body(*refs):
        us = list(refs[0:n_in]) + [None] * pad_in
        lsrc_r, ldst_r, lw_r, z1_r = refs[n_in:n_in + 4]
        ps = list(refs[n_in + 4:n_in + 4 + n_out])
        rest = refs[n_in + 4 + n_out:]
        accs = list(rest[0:n_out])
        li_v, di_v, lw_v = rest[n_out:n_out + 3]
        gs = list(rest[n_out + 3:n_out + 3 + n_in]) + [None] * pad_in
        vs = list(rest[n_out + 3 + n_in:n_out + 3 + n_in + n_out])
        sem, sem2 = rest[n_out + 3 + n_in + n_out:]
        if pairwise:
            ps4 = [ps[0], None, ps[1], None] if n_out == 2 else [ps[0], None, None, None]
            accs4 = [accs[0], None, accs[1], None] if n_out == 2 else [accs[0], None, None, None]
            vs4 = [vs[0], None, vs[1], None] if n_out == 2 else [vs[0], None, None, None]
        else:
            ps4 = ps + [None] * pad_in
            accs4 = accs + [None] * pad_in
            vs4 = vs + [None] * pad_in
        _lprop_body(us[0], us[1], us[2], us[3], lsrc_r, ldst_r, lw_r, z1_r,
                    ps4[0], ps4[1], ps4[2], ps4[3], accs4[0], accs4[1],
                    accs4[2], accs4[3], li_v, di_v, lw_v, gs[0], gs[1], gs[2],
                    gs[3], vs4[0], vs4[1], vs4[2], vs4[3], sem, sem2,
                    pairwise=pairwise)

    f = pl.kernel(
        body,
        out_type=tuple(jax.ShapeDtypeStruct((NC, N_EDGES), jnp.float32)
                       for _ in range(n_out)),
        mesh=_sc_mesh(),
        compiler_params=pltpu.CompilerParams(needs_layout_passes=False, use_tc_tiling_on_sc=False),
        scratch_types=(
            [pltpu.VMEM_SHARED((N_EDGES,), jnp.float32)] * n_out +
            [pltpu.VMEM((_LCHUNK,), jnp.int32),
             pltpu.VMEM((_LCHUNK,), jnp.int32),
             pltpu.VMEM((_LCHUNK,), jnp.float32)] +
            [pltpu.VMEM((_LCHUNK,), jnp.float32)] * (n_in + n_out) +
            [pltpu.SemaphoreType.DMA, pltpu.SemaphoreType.DMA]
        ),
    )
    return f(*ucols, lsrc, ldst, lw, z1)


@jax.jit
def _gcn_call(table, nsrc, ndst, w):
    z128 = jnp.zeros((N_NODES // NS, D), jnp.float32)
    f = pl.kernel(
        _gcn_body,
        out_type=jax.ShapeDtypeStruct((NC, N_NODES, D), jnp.float32),
        mesh=_sc_mesh(),
        compiler_params=pltpu.CompilerParams(needs_layout_passes=False, use_tc_tiling_on_sc=False),
        scratch_types=[
            pltpu.VMEM_SHARED((N_NODES, D), jnp.float32),
            pltpu.VMEM((N_EDGES // (NC * NS),), jnp.int32),
            pltpu.VMEM((N_EDGES // (NC * NS),), jnp.int32),
            pltpu.VMEM((N_EDGES // (NC * NS),), jnp.float32),
            pltpu.VMEM((_ECHUNK, D), jnp.float32),
            pltpu.VMEM((_ECHUNK, D), jnp.float32),
            pltpu.SemaphoreType.DMA,
            pltpu.SemaphoreType.DMA,
        ],
    )
    return f(table, nsrc, ndst, w, z128)


def kernel(h_entity, x_link, edge_index_link, edge_weight_link, edge_index,
           W_edge, b_edge, hl_w0, hl_w1, hl_w2, hl_b, W_cau, b_cau, W_t, b_t,
           gcn_w0, gcn_w1, gcn_w2, gcn_b):
    f32 = jnp.float32
    lsrc = edge_index_link[0]
    ldst = edge_index_link[1]
    nsrc = edge_index[0]
    ndst = edge_index[1]

    Wd = jnp.stack([W_cau[:, 2] - W_cau[:, 3], W_cau[:, 4] - W_cau[:, 5]], axis=1)
    bd = jnp.stack([b_cau[2] - b_cau[3], b_cau[4] - b_cau[5]])
    A0, A1, A2 = hl_w0 @ Wd, hl_w1 @ Wd, hl_w2 @ Wd
    C0 = A0 + A1 + A2
    C12 = jnp.concatenate([A1 + 2.0 * A2, 0.5 * A2], axis=1)
    M8 = jnp.concatenate([W_edge @ C0, W_edge @ C12,
                          jnp.zeros((8, 2), f32)], axis=1)
    b8 = jnp.concatenate([b_edge @ C0 + hl_b @ Wd + bd, b_edge @ C12,
                          jnp.zeros((2,), f32)])[None, :]

    BE = 4000
    t8 = pl.pallas_call(
        _t8_body,
        grid=(N_EDGES // BE,),
        in_specs=[pl.BlockSpec((BE, 8), lambda i: (i, 0)),
                  pl.BlockSpec((8, 8), lambda i: (0, 0)),
                  pl.BlockSpec((1, 8), lambda i: (0, 0))],
        out_specs=pl.BlockSpec((BE, 8), lambda i: (i, 0)),
        out_shape=jax.ShapeDtypeStruct((N_EDGES, 8), f32),
    )(x_link, M8, b8)

    BN = 1000
    he = pl.pallas_call(
        _he_body,
        grid=(N_NODES // BN,),
        in_specs=[pl.BlockSpec((T, BN, D), lambda i: (0, i, 0)),
                  pl.BlockSpec((T, 1), lambda i: (0, 0)),
                  pl.BlockSpec((1, 1), lambda i: (0, 0))],
        out_specs=pl.BlockSpec((BN, D), lambda i: (i, 0)),
        out_shape=jax.ShapeDtypeStruct((N_NODES, D), f32),
    )(h_entity, W_t, b_t[:, None])

    lw = edge_weight_link
    pv = _lprop_call([t8[:, 2], t8[:, 3], t8[:, 4], t8[:, 5]],
                     lsrc, ldst, lw)
    q = _lprop_call([pv[2][0], pv[2][1], pv[3][0], pv[3][1]],
                    lsrc, ldst, lw, pairwise=True)
    w1, w2 = pl.pallas_call(
        _gate2_body,
        out_shape=(jax.ShapeDtypeStruct((N_EDGES,), f32),
                   jax.ShapeDtypeStruct((N_EDGES,), f32)),
    )(t8[:, 0], pv[0], q[0], t8[:, 1], pv[1], q[1])

    h1p = _gcn_call(he, nsrc, ndst, w1)

    BC = 1000
    h1 = pl.pallas_call(
        _combine_body,
        grid=(N_NODES // BC,),
        in_specs=[pl.BlockSpec((2, BC, D), lambda i: (0, i, 0))],
        out_specs=pl.BlockSpec((BC, D), lambda i: (i, 0)),
        out_shape=jax.ShapeDtypeStruct((N_NODES, D), f32),
    )(h1p)

    h2p = _gcn_call(h1, nsrc, ndst, w2)

    BF = 1000
    out2 = pl.pallas_call(
        _final_body,
        grid=(N_NODES // BF,),
        in_specs=[pl.BlockSpec((BF, D), lambda i: (i, 0)),
                  pl.BlockSpec((BF, D), lambda i: (i, 0)),
                  pl.BlockSpec((2, BF, D), lambda i: (0, i, 0)),
                  pl.BlockSpec((D, D), lambda i: (0, 0)),
                  pl.BlockSpec((D, D), lambda i: (0, 0)),
                  pl.BlockSpec((D, D), lambda i: (0, 0)),
                  pl.BlockSpec((1, D), lambda i: (0, 0))],
        out_specs=pl.BlockSpec((BF, D), lambda i: (i, 0)),
        out_shape=jax.ShapeDtypeStruct((N_NODES, D), f32),
    )(he, h1, h2p, gcn_w0, gcn_w1, gcn_w2, gcn_b[None, :])
    return out2


# speedup vs baseline: 7.7389x; 1.7212x over previous
import functools

import jax
import jax.numpy as jnp
from jax import lax
from jax.experimental import pallas as pl
from jax.experimental.pallas import tpu as pltpu
from jax.experimental.pallas import tpu_sc as plsc

N_NODES = 10000
N_EDGES = 320000
N_LL = 640000
D = 128
T = 16

NC = 2
NS = 16
L = 16


def _he_body(h_ref, wt_ref, bt_ref, out_ref):
    acc = h_ref[0] * wt_ref[0, 0]
    for t in range(1, T):
        acc = acc + h_ref[t] * wt_ref[t, 0]
    out_ref[...] = acc + bt_ref[0, 0]


def _t8t_body(x_ref, m_ref, b_ref, out_ref):
    t = lax.dot_general(m_ref[...], x_ref[...], (((0,), (1,)), ((), ())),
                        preferred_element_type=jnp.float32)
    out_ref[...] = t + b_ref[...]


def _combine_body(p_ref, out_ref):
    out_ref[...] = p_ref[0] + p_ref[1]


def _final_body(he_ref, h1_ref, h2p_ref, g0_ref, g1_ref, g2_ref, gb_ref, out_ref):
    f32 = jnp.float32
    acc = jnp.dot(he_ref[...], g0_ref[...], preferred_element_type=f32)
    acc = acc + jnp.dot(h1_ref[...], g1_ref[...], preferred_element_type=f32)
    h2 = h2p_ref[0] + h2p_ref[1]
    acc = acc + jnp.dot(h2, g2_ref[...], preferred_element_type=f32)
    out_ref[...] = acc + gb_ref[...]


_LCHUNK = 4000
_ECHUNK = 80
_RCHUNK = 2000


def _lprop_body(u0_hbm, u1_hbm, u2_hbm, u3_hbm,
                lsrc_hbm, ldst_hbm, lw_hbm, z1_hbm,
                p0_hbm, p1_hbm, p2_hbm, p3_hbm,
                acc0, acc1, acc2, acc3,
                li_v, di_v, lw_v, g0_v, g1_v, g2_v, g3_v,
                v0_v, v1_v, v2_v, v3_v, sem, sem2, pairwise=False):
    c = lax.axis_index("c")
    s = lax.axis_index("s")
    cols = [(u, p, a, g, v) for u, p, a, g, v in
            ((u0_hbm, p0_hbm, acc0, g0_v, v0_v),
             (u1_hbm, p1_hbm, acc1, g1_v, v1_v),
             (u2_hbm, p2_hbm, acc2, g2_v, v2_v),
             (u3_hbm, p3_hbm, acc3, g3_v, v3_v)) if u is not None]

    zrows = N_EDGES // NS
    for col in cols:
        if col[2] is not None:
            pltpu.sync_copy(z1_hbm, col[2].at[pl.ds(s * zrows, zrows)])
    plsc.subcore_barrier()

    links_per_tile = N_LL // (NC * NS)
    nchunks = links_per_tile // _LCHUNK
    nv = _LCHUNK // L

    def chunk(j, carry):
        base = (c * NS + s) * links_per_tile + j * _LCHUNK
        pltpu.sync_copy(lsrc_hbm.at[pl.ds(base, _LCHUNK)], li_v)
        pltpu.sync_copy(ldst_hbm.at[pl.ds(base, _LCHUNK)], di_v)
        pltpu.sync_copy(lw_hbm.at[pl.ds(base, _LCHUNK)], lw_v)
        descs = [pltpu.async_copy(col[0].at[li_v], col[3], sem)
                 for col in cols]
        sdescs = []
        if pairwise:
            for k in range(len(cols) // 2):
                ca, cb = cols[2 * k], cols[2 * k + 1]
                descs[2 * k].wait()
                descs[2 * k + 1].wait()

                def mul(jj, c2, ga=ca[3], gb=cb[3], v=ca[4]):
                    sl = pl.ds(jj * L, L)
                    v[sl] = (ga[sl] + gb[sl]) * lw_v[sl]
                    return c2

                lax.fori_loop(0, nv, mul, 0)
                sdescs.append(pltpu.async_copy(ca[4], ca[2].at[di_v], sem2,
                                               add=True))
        else:
            for k, col in enumerate(cols):
                a, g, v = col[2], col[3], col[4]
                descs[k].wait()

                def mul(jj, c2, g=g, v=v):
                    sl = pl.ds(jj * L, L)
                    v[sl] = g[sl] * lw_v[sl]
                    return c2

                lax.fori_loop(0, nv, mul, 0)
                sdescs.append(pltpu.async_copy(v, a.at[di_v], sem2, add=True))
        for dd in sdescs:
            dd.wait()
        return carry

    lax.fori_loop(0, nchunks, chunk, 0)
    plsc.subcore_barrier()
    for col in cols:
        if col[2] is not None:
            pltpu.sync_copy(col[2].at[pl.ds(s * zrows, zrows)],
                            col[1].at[c, pl.ds(s * zrows, zrows)])


def _gate2_body(td0, v0, q0, td1, v1, q1, w1o, w2o):
    d0 = td0[...] - (v0[0] + v0[1]) + q0[0] + q0[1]
    w1o[...] = 1.0 / (1.0 + jnp.exp(-d0))
    d1 = td1[...] - (v1[0] + v1[1]) + q1[0] + q1[1]
    w2o[...] = 1.0 / (1.0 + jnp.exp(-d1))


def _gcn_body(table_hbm, nsrc_hbm, ndst_hbm, w_hbm, z128_hbm,
              part_hbm,
              acc, sib_v, dib_v, wb_v,
              rows0_v, rows1_v, sem0, sem1):
    c = lax.axis_index("c")
    s = lax.axis_index("s")
    iota = lax.iota(jnp.int32, L)
    rows = (rows0_v, rows1_v)
    sems = (sem0, sem1)

    zrows = N_NODES // NS
    pltpu.sync_copy(z128_hbm, acc.at[pl.ds(s * zrows, zrows)])
    plsc.subcore_barrier()

    edges_per_tile = N_EDGES // (NC * NS)
    nchunks = edges_per_tile // _ECHUNK
    base0 = (c * NS + s) * edges_per_tile

    pltpu.sync_copy(nsrc_hbm.at[pl.ds(base0, edges_per_tile)], sib_v)
    pltpu.sync_copy(ndst_hbm.at[pl.ds(base0, edges_per_tile)], dib_v)
    pltpu.sync_copy(w_hbm.at[pl.ds(base0, edges_per_tile)], wb_v)

    def gather(j, b):
        return pltpu.async_copy(
            table_hbm.at[sib_v.at[pl.ds(j * _ECHUNK, _ECHUNK)]], rows[b],
            sems[b])

    desc = {0: gather(0, 0)}
    for j in range(nchunks):
        b = j % 2
        if j + 1 < nchunks:
            nb = (j + 1) % 2
            desc[nb] = gather(j + 1, nb)
        desc[b].wait()

        def scale(i, _2, j=j, b=b):
            wsplat = plsc.load_gather(wb_v, [iota * 0 + (j * _ECHUNK + i)])
            for k in range(D // L):
                rows[b][i, pl.ds(k * L, L)] = rows[b][i, pl.ds(k * L, L)] * wsplat
            return _2

        lax.fori_loop(0, _ECHUNK, scale, 0)
        pltpu.sync_copy(rows[b], acc.at[dib_v.at[pl.ds(j * _ECHUNK, _ECHUNK)]],
                        add=True)

    plsc.subcore_barrier()
    pltpu.sync_copy(acc.at[pl.ds(s * zrows, zrows)],
                    part_hbm.at[c, pl.ds(s * zrows, zrows)])


def _sc_mesh():
    return plsc.VectorSubcoreMesh(core_axis_name="c", subcore_axis_name="s",
                                  num_cores=NC, num_subcores=NS)


def _lprop_call(ucols, lsrc, ldst, lw, pairwise=False):
    n_in = len(ucols)
    n_out = n_in // 2 if pairwise else n_in
    pad_in = 4 - n_in
    z1 = jnp.zeros((N_EDGES // NS,), jnp.float32)

    def body(*refs):
        us = list(refs[0:n_in]) + [None] * pad_in
        lsrc_r, ldst_r, lw_r, z1_r = refs[n_in:n_in + 4]
        ps = list(refs[n_in + 4:n_in + 4 + n_out])
        rest = refs[n_in + 4 + n_out:]
        accs = list(rest[0:n_out])
        li_v, di_v, lw_v = rest[n_out:n_out + 3]
        gs = list(rest[n_out + 3:n_out + 3 + n_in]) + [None] * pad_in
        vs = list(rest[n_out + 3 + n_in:n_out + 3 + n_in + n_out])
        sem, sem2 = rest[n_out + 3 + n_in + n_out:]
        if pairwise:
            ps4 = [ps[0], None, ps[1], None] if n_out == 2 else [ps[0], None, None, None]
            accs4 = [accs[0], None, accs[1], None] if n_out == 2 else [accs[0], None, None, None]
            vs4 = [vs[0], None, vs[1], None] if n_out == 2 else [vs[0], None, None, None]
        else:
            ps4 = ps + [None] * pad_in
            accs4 = accs + [None] * pad_in
            vs4 = vs + [None] * pad_in
        _lprop_body(us[0], us[1], us[2], us[3], lsrc_r, ldst_r, lw_r, z1_r,
                    ps4[0], ps4[1], ps4[2], ps4[3], accs4[0], accs4[1],
                    accs4[2], accs4[3], li_v, di_v, lw_v, gs[0], gs[1], gs[2],
                    gs[3], vs4[0], vs4[1], vs4[2], vs4[3], sem, sem2,
                    pairwise=pairwise)

    f = pl.kernel(
        body,
        out_type=tuple(jax.ShapeDtypeStruct((NC, N_EDGES), jnp.float32)
                       for _ in range(n_out)),
        mesh=_sc_mesh(),
        compiler_params=pltpu.CompilerParams(needs_layout_passes=False, use_tc_tiling_on_sc=False),
        scratch_types=(
            [pltpu.VMEM_SHARED((N_EDGES,), jnp.float32)] * n_out +
            [pltpu.VMEM((_LCHUNK,), jnp.int32),
             pltpu.VMEM((_LCHUNK,), jnp.int32),
             pltpu.VMEM((_LCHUNK,), jnp.float32)] +
            [pltpu.VMEM((_LCHUNK,), jnp.float32)] * (n_in + n_out) +
            [pltpu.SemaphoreType.DMA, pltpu.SemaphoreType.DMA]
        ),
    )
    return f(*ucols, lsrc, ldst, lw, z1)


@jax.jit
def _gcn_call(table, nsrc, ndst, w):
    z128 = jnp.zeros((N_NODES // NS, D), jnp.float32)
    f = pl.kernel(
        _gcn_body,
        out_type=jax.ShapeDtypeStruct((NC, N_NODES, D), jnp.float32),
        mesh=_sc_mesh(),
        compiler_params=pltpu.CompilerParams(needs_layout_passes=False, use_tc_tiling_on_sc=False),
        scratch_types=[
            pltpu.VMEM_SHARED((N_NODES, D), jnp.float32),
            pltpu.VMEM((N_EDGES // (NC * NS),), jnp.int32),
            pltpu.VMEM((N_EDGES // (NC * NS),), jnp.int32),
            pltpu.VMEM((N_EDGES // (NC * NS),), jnp.float32),
            pltpu.VMEM((_ECHUNK, D), jnp.float32),
            pltpu.VMEM((_ECHUNK, D), jnp.float32),
            pltpu.SemaphoreType.DMA,
            pltpu.SemaphoreType.DMA,
        ],
    )
    return f(table, nsrc, ndst, w, z128)


def kernel(h_entity, x_link, edge_index_link, edge_weight_link, edge_index,
           W_edge, b_edge, hl_w0, hl_w1, hl_w2, hl_b, W_cau, b_cau, W_t, b_t,
           gcn_w0, gcn_w1, gcn_w2, gcn_b):
    f32 = jnp.float32
    lsrc = edge_index_link[0]
    ldst = edge_index_link[1]
    nsrc = edge_index[0]
    ndst = edge_index[1]

    Wd = jnp.stack([W_cau[:, 2] - W_cau[:, 3], W_cau[:, 4] - W_cau[:, 5]], axis=1)
    bd = jnp.stack([b_cau[2] - b_cau[3], b_cau[4] - b_cau[5]])
    A0, A1, A2 = hl_w0 @ Wd, hl_w1 @ Wd, hl_w2 @ Wd
    C0 = A0 + A1 + A2
    C12 = jnp.concatenate([A1 + 2.0 * A2, 0.5 * A2], axis=1)
    M8 = jnp.concatenate([W_edge @ C0, W_edge @ C12,
                          jnp.zeros((8, 2), f32)], axis=1)
    b8 = jnp.concatenate([b_edge @ C0 + hl_b @ Wd + bd, b_edge @ C12,
                          jnp.zeros((2,), f32)])[:, None]

    BE = 3200
    t8t = pl.pallas_call(
        _t8t_body,
        grid=(N_EDGES // BE,),
        in_specs=[pl.BlockSpec((BE, 8), lambda i: (i, 0)),
                  pl.BlockSpec((8, 8), lambda i: (0, 0)),
                  pl.BlockSpec((8, 1), lambda i: (0, 0))],
        out_specs=pl.BlockSpec((8, BE), lambda i: (0, i)),
        out_shape=jax.ShapeDtypeStruct((8, N_EDGES), f32),
    )(x_link, M8, b8)

    BN = 1000
    he = pl.pallas_call(
        _he_body,
        grid=(N_NODES // BN,),
        in_specs=[pl.BlockSpec((T, BN, D), lambda i: (0, i, 0)),
                  pl.BlockSpec((T, 1), lambda i: (0, 0)),
                  pl.BlockSpec((1, 1), lambda i: (0, 0))],
        out_specs=pl.BlockSpec((BN, D), lambda i: (i, 0)),
        out_shape=jax.ShapeDtypeStruct((N_NODES, D), f32),
    )(h_entity, W_t, b_t[:, None])

    lw = edge_weight_link
    pv = _lprop_call([t8t[2], t8t[3], t8t[4], t8t[5]],
                     lsrc, ldst, lw)
    q = _lprop_call([pv[2][0], pv[2][1], pv[3][0], pv[3][1]],
                    lsrc, ldst, lw, pairwise=True)
    w1, w2 = pl.pallas_call(
        _gate2_body,
        out_shape=(jax.ShapeDtypeStruct((N_EDGES,), f32),
                   jax.ShapeDtypeStruct((N_EDGES,), f32)),
    )(t8t[0], pv[0], q[0], t8t[1], pv[1], q[1])

    h1p = _gcn_call(he, nsrc, ndst, w1)

    BC = 1000
    h1 = pl.pallas_call(
        _combine_body,
        grid=(N_NODES // BC,),
        in_specs=[pl.BlockSpec((2, BC, D), lambda i: (0, i, 0))],
        out_specs=pl.BlockSpec((BC, D), lambda i: (i, 0)),
        out_shape=jax.ShapeDtypeStruct((N_NODES, D), f32),
    )(h1p)

    h2p = _gcn_call(h1, nsrc, ndst, w2)

    BF = 1000
    out2 = pl.pallas_call(
        _final_body,
        grid=(N_NODES // BF,),
        in_specs=[pl.BlockSpec((BF, D), lambda i: (i, 0)),
                  pl.BlockSpec((BF, D), lambda i: (i, 0)),
                  pl.BlockSpec((2, BF, D), lambda i: (0, i, 0)),
                  pl.BlockSpec((D, D), lambda i: (0, 0)),
                  pl.BlockSpec((D, D), lambda i: (0, 0)),
                  pl.BlockSpec((D, D), lambda i: (0, 0)),
                  pl.BlockSpec((1, D), lambda i: (0, 0))],
        out_specs=pl.BlockSpec((BF, D), lambda i: (i, 0)),
        out_shape=jax.ShapeDtypeStruct((N_NODES, D), f32),
    )(he, h1, h2p, gcn_w0, gcn_w1, gcn_w2, gcn_b[None, :])
    return out2
